# baseline (speedup 1.0000x reference)
import functools

import jax
import jax.numpy as jnp
from jax import lax
from jax.experimental import pallas as pl
from jax.experimental.pallas import tpu as pltpu
from jax.experimental.pallas import tpu_sc as plsc

N = 10000
E = 320000
D = 128

NPAD = 10112
AROWS_PER_TILE = NPAD // 16
CHUNK = 128
CH0 = 138
CH1 = 20
TOT_CH = 16 * (CH0 + CH1)
E_PAD = TOT_CH * CHUNK

NBUF = 3


_sc_mesh = plsc.VectorSubcoreMesh(core_axis_name="c", subcore_axis_name="s")


@functools.partial(
    pl.kernel,
    out_type=jax.ShapeDtypeStruct((2, NPAD, D), jnp.float32),
    mesh=_sc_mesh,
    scratch_types=[
        pltpu.VMEM((NBUF, 2, CHUNK), jnp.int32),
        pltpu.VMEM((NBUF, CHUNK, D), jnp.float32),
        pltpu.VMEM_SHARED((NPAD, D), jnp.float32),
        pltpu.SemaphoreType.DMA,
        pltpu.SemaphoreType.DMA,
        pltpu.SemaphoreType.DMA,
    ],
)
def _sc_aggregate(h_hbm, eidx_hbm, zeros_hbm, out_hbm,
                  eidx, rows, acc, sem_g, sem_s, sem_i):
    c = lax.axis_index("c")
    s = lax.axis_index("s")
    n_ch = jnp.where(c == 0, CH0, CH1)
    cbase = jnp.where(c == 0, s * CH0, 16 * CH0 + s * CH1)
    row0 = s * AROWS_PER_TILE

    pltpu.sync_copy(zeros_hbm, acc.at[pl.ds(row0, AROWS_PER_TILE)])
    plsc.subcore_barrier()

    def idx_start(j):
        pltpu.async_copy(eidx_hbm.at[cbase + j], eidx.at[lax.rem(j, NBUF)],
                         sem_i)

    def idx_wait(j):
        pltpu.make_async_copy(eidx_hbm.at[cbase + j],
                              eidx.at[lax.rem(j, NBUF)], sem_i).wait()

    def gather_start(j):
        pltpu.async_copy(h_hbm.at[eidx.at[lax.rem(j, NBUF), 0]],
                         rows.at[lax.rem(j, NBUF)], sem_g)

    def gather_wait(j):
        pltpu.make_async_copy(h_hbm.at[eidx.at[lax.rem(j, NBUF), 0]],
                              rows.at[lax.rem(j, NBUF)], sem_g).wait()

    def scatter_start(j):
        pltpu.async_copy(rows.at[lax.rem(j, NBUF)],
                         acc.at[eidx.at[lax.rem(j, NBUF), 1]],
                         sem_s, add=True)

    def scatter_wait(j):
        pltpu.make_async_copy(rows.at[lax.rem(j, NBUF)],
                              acc.at[eidx.at[lax.rem(j, NBUF), 1]],
                              sem_s).wait()

    idx_start(0)
    idx_wait(0)
    gather_start(0)

    def chunk_body(j, carry):
        @pl.when(j > 1)
        def _():
            scatter_wait(j - 2)

        @pl.when(j + 1 < n_ch)
        def _():
            idx_start(j + 1)

        gather_wait(j)
        scatter_start(j)

        @pl.when(j + 1 < n_ch)
        def _():
            idx_wait(j + 1)
            gather_start(j + 1)

        return carry

    lax.fori_loop(0, n_ch, chunk_body, 0)
    scatter_wait(n_ch - 2)
    scatter_wait(n_ch - 1)
    plsc.subcore_barrier()

    pltpu.sync_copy(acc.at[pl.ds(row0, AROWS_PER_TILE)],
                    out_hbm.at[c, pl.ds(row0, AROWS_PER_TILE)])


_BLK = 632


def _lin1_body(x_ref, w_ref, b_ref, o_ref):
    o_ref[...] = lax.dot_general(
        x_ref[...], w_ref[...], (((1,), (1,)), ((), ())),
        preferred_element_type=jnp.float32) + b_ref[...]


def _lin1(xp, W, b):
    return pl.pallas_call(
        _lin1_body,
        grid=(16,),
        in_specs=[
            pl.BlockSpec((_BLK, D), lambda i: (i, 0)),
            pl.BlockSpec((D, D), lambda i: (0, 0)),
            pl.BlockSpec((1, D), lambda i: (0, 0)),
        ],
        out_specs=pl.BlockSpec((_BLK, D), lambda i: (i, 0)),
        out_shape=jax.ShapeDtypeStruct((NPAD, D), jnp.float32),
    )(xp, W, b.reshape(1, D))


def _lin2_body(v_ref, n_ref, w_ref, b_ref, o_ref):
    t = jnp.maximum(n_ref[...] * (v_ref[0] + v_ref[1]), 0.0)
    o_ref[...] = lax.dot_general(
        t, w_ref[...], (((1,), (1,)), ((), ())),
        preferred_element_type=jnp.float32) + b_ref[...]


def _lin2(v, normp, W, b):
    return pl.pallas_call(
        _lin2_body,
        grid=(16,),
        in_specs=[
            pl.BlockSpec((2, _BLK, D), lambda i: (0, i, 0)),
            pl.BlockSpec((_BLK, 1), lambda i: (i, 0)),
            pl.BlockSpec((D, D), lambda i: (0, 0)),
            pl.BlockSpec((1, D), lambda i: (0, 0)),
        ],
        out_specs=pl.BlockSpec((_BLK, D), lambda i: (i, 0)),
        out_shape=jax.ShapeDtypeStruct((NPAD, D), jnp.float32),
    )(v, normp, W, b.reshape(1, D))


def _final_body(v_ref, n_ref, o_ref):
    o_ref[...] = n_ref[...] * (v_ref[0] + v_ref[1])


def _final(v, normp):
    return pl.pallas_call(
        _final_body,
        grid=(16,),
        in_specs=[
            pl.BlockSpec((2, _BLK, D), lambda i: (0, i, 0)),
            pl.BlockSpec((_BLK, 1), lambda i: (i, 0)),
        ],
        out_specs=pl.BlockSpec((_BLK, D), lambda i: (i, 0)),
        out_shape=jax.ShapeDtypeStruct((NPAD, D), jnp.float32),
    )(v, normp)


def kernel(x, norm, edge_index, W1, b1, W2, b2):
    xp = jnp.pad(x, ((0, NPAD - N), (0, 0)))
    normp = jnp.pad(norm, ((0, NPAD - N), (0, 0)))
    src = jnp.pad(edge_index[0].astype(jnp.int32),
                  (0, E_PAD - E)).reshape(TOT_CH, 1, CHUNK)
    dst = jnp.pad(edge_index[1].astype(jnp.int32), (0, E_PAD - E),
                  constant_values=N
                  ).reshape(TOT_CH, 1, CHUNK)
    eidx = jnp.concatenate([src, dst], axis=1)
    zeros = jnp.zeros((AROWS_PER_TILE, D), jnp.float32)

    h1 = _lin1(xp, W1, b1)
    v1 = _sc_aggregate(h1, eidx, zeros)
    h2 = _lin2(v1, normp, W2, b2)
    v2 = _sc_aggregate(h2, eidx, zeros)
    out = _final(v2, normp)
    return out[:N]


# speedup vs baseline: 1.0443x; 1.0443x over previous
import functools

import jax
import jax.numpy as jnp
from jax import lax
from jax.experimental import pallas as pl
from jax.experimental.pallas import tpu as pltpu
from jax.experimental.pallas import tpu_sc as plsc

N = 10000
E = 320000
D = 128

NPAD = 10112
AROWS_PER_TILE = NPAD // 16
CHUNK = 128
CH0 = 130
CH1 = 28
TOT_CH = 16 * (CH0 + CH1)
E_PAD = TOT_CH * CHUNK

NBUF = 3


_sc_mesh = plsc.VectorSubcoreMesh(core_axis_name="c", subcore_axis_name="s")


@functools.partial(
    pl.kernel,
    out_type=jax.ShapeDtypeStruct((2, NPAD, D), jnp.float32),
    mesh=_sc_mesh,
    scratch_types=[
        pltpu.VMEM((NBUF, 2, CHUNK), jnp.int32),
        pltpu.VMEM((NBUF, CHUNK, D), jnp.float32),
        pltpu.VMEM_SHARED((NPAD, D), jnp.float32),
        pltpu.SemaphoreType.DMA,
        pltpu.SemaphoreType.DMA,
        pltpu.SemaphoreType.DMA,
    ],
)
def _sc_aggregate(h_hbm, eidx_hbm, zeros_hbm, out_hbm,
                  eidx, rows, acc, sem_g, sem_s, sem_i):
    c = lax.axis_index("c")
    s = lax.axis_index("s")
    n_ch = jnp.where(c == 0, CH0, CH1)
    cbase = jnp.where(c == 0, s * CH0, 16 * CH0 + s * CH1)
    row0 = s * AROWS_PER_TILE

    pltpu.sync_copy(zeros_hbm, acc.at[pl.ds(row0, AROWS_PER_TILE)])
    plsc.subcore_barrier()

    def idx_start(j):
        pltpu.async_copy(eidx_hbm.at[cbase + j], eidx.at[lax.rem(j, NBUF)],
                         sem_i)

    def idx_wait(j):
        pltpu.make_async_copy(eidx_hbm.at[cbase + j],
                              eidx.at[lax.rem(j, NBUF)], sem_i).wait()

    def gather_start(j):
        pltpu.async_copy(h_hbm.at[eidx.at[lax.rem(j, NBUF), 0]],
                         rows.at[lax.rem(j, NBUF)], sem_g)

    def gather_wait(j):
        pltpu.make_async_copy(h_hbm.at[eidx.at[lax.rem(j, NBUF), 0]],
                              rows.at[lax.rem(j, NBUF)], sem_g).wait()

    def scatter_start(j):
        pltpu.async_copy(rows.at[lax.rem(j, NBUF)],
                         acc.at[eidx.at[lax.rem(j, NBUF), 1]],
                         sem_s, add=True)

    def scatter_wait(j):
        pltpu.make_async_copy(rows.at[lax.rem(j, NBUF)],
                              acc.at[eidx.at[lax.rem(j, NBUF), 1]],
                              sem_s).wait()

    idx_start(0)
    idx_wait(0)
    gather_start(0)

    def chunk_body(j, carry):
        @pl.when(j > 1)
        def _():
            scatter_wait(j - 2)

        @pl.when(j + 1 < n_ch)
        def _():
            idx_start(j + 1)

        gather_wait(j)
        scatter_start(j)

        @pl.when(j + 1 < n_ch)
        def _():
            idx_wait(j + 1)
            gather_start(j + 1)

        return carry

    lax.fori_loop(0, n_ch, chunk_body, 0)
    scatter_wait(n_ch - 2)
    scatter_wait(n_ch - 1)
    plsc.subcore_barrier()

    pltpu.sync_copy(acc.at[pl.ds(row0, AROWS_PER_TILE)],
                    out_hbm.at[c, pl.ds(row0, AROWS_PER_TILE)])


_BLK = 632


def _lin1_body(x_ref, w_ref, b_ref, o_ref):
    o_ref[...] = lax.dot_general(
        x_ref[...], w_ref[...], (((1,), (1,)), ((), ())),
        preferred_element_type=jnp.float32) + b_ref[...]


def _lin1(xp, W, b):
    return pl.pallas_call(
        _lin1_body,
        grid=(16,),
        in_specs=[
            pl.BlockSpec((_BLK, D), lambda i: (i, 0)),
            pl.BlockSpec((D, D), lambda i: (0, 0)),
            pl.BlockSpec((1, D), lambda i: (0, 0)),
        ],
        out_specs=pl.BlockSpec((_BLK, D), lambda i: (i, 0)),
        out_shape=jax.ShapeDtypeStruct((NPAD, D), jnp.float32),
    )(xp, W, b.reshape(1, D))


def _lin2_body(v_ref, n_ref, w_ref, b_ref, o_ref):
    t = jnp.maximum(n_ref[...] * (v_ref[0] + v_ref[1]), 0.0)
    o_ref[...] = lax.dot_general(
        t, w_ref[...], (((1,), (1,)), ((), ())),
        preferred_element_type=jnp.float32) + b_ref[...]


def _lin2(v, normp, W, b):
    return pl.pallas_call(
        _lin2_body,
        grid=(16,),
        in_specs=[
            pl.BlockSpec((2, _BLK, D), lambda i: (0, i, 0)),
            pl.BlockSpec((_BLK, 1), lambda i: (i, 0)),
            pl.BlockSpec((D, D), lambda i: (0, 0)),
            pl.BlockSpec((1, D), lambda i: (0, 0)),
        ],
        out_specs=pl.BlockSpec((_BLK, D), lambda i: (i, 0)),
        out_shape=jax.ShapeDtypeStruct((NPAD, D), jnp.float32),
    )(v, normp, W, b.reshape(1, D))


def _final_body(v_ref, n_ref, o_ref):
    o_ref[...] = n_ref[...] * (v_ref[0] + v_ref[1])


def _final(v, normp):
    return pl.pallas_call(
        _final_body,
        grid=(16,),
        in_specs=[
            pl.BlockSpec((2, _BLK, D), lambda i: (0, i, 0)),
            pl.BlockSpec((_BLK, 1), lambda i: (i, 0)),
        ],
        out_specs=pl.BlockSpec((_BLK, D), lambda i: (i, 0)),
        out_shape=jax.ShapeDtypeStruct((NPAD, D), jnp.float32),
    )(v, normp)


def kernel(x, norm, edge_index, W1, b1, W2, b2):
    xp = jnp.pad(x, ((0, NPAD - N), (0, 0)))
    normp = jnp.pad(norm, ((0, NPAD - N), (0, 0)))
    src = jnp.pad(edge_index[0].astype(jnp.int32),
                  (0, E_PAD - E)).reshape(TOT_CH, 1, CHUNK)
    dst = jnp.pad(edge_index[1].astype(jnp.int32), (0, E_PAD - E),
                  constant_values=N
                  ).reshape(TOT_CH, 1, CHUNK)
    eidx = jnp.concatenate([src, dst], axis=1)
    zeros = jnp.zeros((AROWS_PER_TILE, D), jnp.float32)

    h1 = _lin1(xp, W1, b1)
    v1 = _sc_aggregate(h1, eidx, zeros)
    h2 = _lin2(v1, normp, W2, b2)
    v2 = _sc_aggregate(h2, eidx, zeros)
    out = _final(v2, normp)
    return out[:N]


# speedup vs baseline: 1.0507x; 1.0061x over previous
import functools

import jax
import jax.numpy as jnp
from jax import lax
from jax.experimental import pallas as pl
from jax.experimental.pallas import tpu as pltpu
from jax.experimental.pallas import tpu_sc as plsc

N = 10000
E = 320000
D = 128

NPAD = 10112
AROWS_PER_TILE = NPAD // 16
CHUNK = 128
CH0 = 126
CH1 = 32
TOT_CH = 16 * (CH0 + CH1)
E_PAD = TOT_CH * CHUNK

NBUF = 3


_sc_mesh = plsc.VectorSubcoreMesh(core_axis_name="c", subcore_axis_name="s")


@functools.partial(
    pl.kernel,
    out_type=jax.ShapeDtypeStruct((2, NPAD, D), jnp.float32),
    mesh=_sc_mesh,
    scratch_types=[
        pltpu.VMEM((NBUF, 2, CHUNK), jnp.int32),
        pltpu.VMEM((NBUF, CHUNK, D), jnp.float32),
        pltpu.VMEM_SHARED((NPAD, D), jnp.float32),
        pltpu.SemaphoreType.DMA,
        pltpu.SemaphoreType.DMA,
        pltpu.SemaphoreType.DMA,
    ],
)
def _sc_aggregate(h_hbm, eidx_hbm, zeros_hbm, out_hbm,
                  eidx, rows, acc, sem_g, sem_s, sem_i):
    c = lax.axis_index("c")
    s = lax.axis_index("s")
    n_ch = jnp.where(c == 0, CH0, CH1)
    cbase = jnp.where(c == 0, s * CH0, 16 * CH0 + s * CH1)
    row0 = s * AROWS_PER_TILE

    pltpu.sync_copy(zeros_hbm, acc.at[pl.ds(row0, AROWS_PER_TILE)])
    plsc.subcore_barrier()

    def idx_start(j):
        pltpu.async_copy(eidx_hbm.at[cbase + j], eidx.at[lax.rem(j, NBUF)],
                         sem_i)

    def idx_wait(j):
        pltpu.make_async_copy(eidx_hbm.at[cbase + j],
                              eidx.at[lax.rem(j, NBUF)], sem_i).wait()

    def gather_start(j):
        pltpu.async_copy(h_hbm.at[eidx.at[lax.rem(j, NBUF), 0]],
                         rows.at[lax.rem(j, NBUF)], sem_g)

    def gather_wait(j):
        pltpu.make_async_copy(h_hbm.at[eidx.at[lax.rem(j, NBUF), 0]],
                              rows.at[lax.rem(j, NBUF)], sem_g).wait()

    def scatter_start(j):
        pltpu.async_copy(rows.at[lax.rem(j, NBUF)],
                         acc.at[eidx.at[lax.rem(j, NBUF), 1]],
                         sem_s, add=True)

    def scatter_wait(j):
        pltpu.make_async_copy(rows.at[lax.rem(j, NBUF)],
                              acc.at[eidx.at[lax.rem(j, NBUF), 1]],
                              sem_s).wait()

    idx_start(0)
    idx_wait(0)
    gather_start(0)

    def chunk_body(j, carry):
        @pl.when(j > 1)
        def _():
            scatter_wait(j - 2)

        @pl.when(j + 1 < n_ch)
        def _():
            idx_start(j + 1)

        gather_wait(j)
        scatter_start(j)

        @pl.when(j + 1 < n_ch)
        def _():
            idx_wait(j + 1)
            gather_start(j + 1)

        return carry

    lax.fori_loop(0, n_ch, chunk_body, 0)
    scatter_wait(n_ch - 2)
    scatter_wait(n_ch - 1)
    plsc.subcore_barrier()

    pltpu.sync_copy(acc.at[pl.ds(row0, AROWS_PER_TILE)],
                    out_hbm.at[c, pl.ds(row0, AROWS_PER_TILE)])


_BLK = 632


def _lin1_body(x_ref, w_ref, b_ref, o_ref):
    o_ref[...] = lax.dot_general(
        x_ref[...], w_ref[...], (((1,), (1,)), ((), ())),
        preferred_element_type=jnp.float32) + b_ref[...]


def _lin1(xp, W, b):
    return pl.pallas_call(
        _lin1_body,
        grid=(16,),
        in_specs=[
            pl.BlockSpec((_BLK, D), lambda i: (i, 0)),
            pl.BlockSpec((D, D), lambda i: (0, 0)),
            pl.BlockSpec((1, D), lambda i: (0, 0)),
        ],
        out_specs=pl.BlockSpec((_BLK, D), lambda i: (i, 0)),
        out_shape=jax.ShapeDtypeStruct((NPAD, D), jnp.float32),
    )(xp, W, b.reshape(1, D))


def _lin2_body(v_ref, n_ref, w_ref, b_ref, o_ref):
    t = jnp.maximum(n_ref[...] * (v_ref[0] + v_ref[1]), 0.0)
    o_ref[...] = lax.dot_general(
        t, w_ref[...], (((1,), (1,)), ((), ())),
        preferred_element_type=jnp.float32) + b_ref[...]


def _lin2(v, normp, W, b):
    return pl.pallas_call(
        _lin2_body,
        grid=(16,),
        in_specs=[
            pl.BlockSpec((2, _BLK, D), lambda i: (0, i, 0)),
            pl.BlockSpec((_BLK, 1), lambda i: (i, 0)),
            pl.BlockSpec((D, D), lambda i: (0, 0)),
            pl.BlockSpec((1, D), lambda i: (0, 0)),
        ],
        out_specs=pl.BlockSpec((_BLK, D), lambda i: (i, 0)),
        out_shape=jax.ShapeDtypeStruct((NPAD, D), jnp.float32),
    )(v, normp, W, b.reshape(1, D))


def _final_body(v_ref, n_ref, o_ref):
    o_ref[...] = n_ref[...] * (v_ref[0] + v_ref[1])


def _final(v, normp):
    return pl.pallas_call(
        _final_body,
        grid=(16,),
        in_specs=[
            pl.BlockSpec((2, _BLK, D), lambda i: (0, i, 0)),
            pl.BlockSpec((_BLK, 1), lambda i: (i, 0)),
        ],
        out_specs=pl.BlockSpec((_BLK, D), lambda i: (i, 0)),
        out_shape=jax.ShapeDtypeStruct((NPAD, D), jnp.float32),
    )(v, normp)


def kernel(x, norm, edge_index, W1, b1, W2, b2):
    xp = jnp.pad(x, ((0, NPAD - N), (0, 0)))
    normp = jnp.pad(norm, ((0, NPAD - N), (0, 0)))
    src = jnp.pad(edge_index[0].astype(jnp.int32),
                  (0, E_PAD - E)).reshape(TOT_CH, 1, CHUNK)
    dst = jnp.pad(edge_index[1].astype(jnp.int32), (0, E_PAD - E),
                  constant_values=N
                  ).reshape(TOT_CH, 1, CHUNK)
    eidx = jnp.concatenate([src, dst], axis=1)
    zeros = jnp.zeros((AROWS_PER_TILE, D), jnp.float32)

    h1 = _lin1(xp, W1, b1)
    v1 = _sc_aggregate(h1, eidx, zeros)
    h2 = _lin2(v1, normp, W2, b2)
    v2 = _sc_aggregate(h2, eidx, zeros)
    out = _final(v2, normp)
    return out[:N]
